# Initial kernel scaffold; baseline (speedup 1.0000x reference)
import jax
import jax.numpy as jnp
from jax.experimental import pallas as pl


def kernel(context, nodes, nodes_mask, turns_coeffs, table):
    raise NotImplementedError("write your pallas kernel here")


# speedup vs baseline: 1.4983x; 1.4983x over previous
import dataclasses
import functools

import jax
import jax.numpy as jnp
from jax import lax
from jax.experimental import pallas as pl
from jax.experimental.pallas import tpu as pltpu
from jax.experimental.pallas import tpu_sc as plsc

B = 16384
CTX = 20
PATH = 20
D = 32
L = 16
NW = 32
GB = 4
NBLK = B // GB
BLK_PER_W = NBLK // NW
ROWS = GB * CTX


def _rsqrt_nr(n2):
    i = lax.bitcast_convert_type(n2, jnp.int32)
    i = jnp.int32(0x5F3759DF) - lax.shift_right_logical(i, 1)
    y = lax.bitcast_convert_type(i, jnp.float32)
    for _ in range(3):
        y = y * (1.5 - 0.5 * n2 * y * y)
    return y


def _row_scale(v0, v1):
    n2 = jnp.sum(v0 * v0 + v1 * v1)
    return jnp.where(n2 > 1.0, _rsqrt_nr(n2), 1.0)


def _sc_excitations(table, ctx_idx, node_idx):
    mesh = plsc.VectorSubcoreMesh(core_axis_name="c", subcore_axis_name="s")
    cp = pltpu.CompilerParams()
    fields = pltpu.CompilerParams.__dataclass_fields__
    if "needs_layout_passes" in fields:
        cp = dataclasses.replace(cp, needs_layout_passes=False)
    if "use_tc_tiling_on_sc" in fields:
        cp = dataclasses.replace(cp, use_tc_tiling_on_sc=False)

    @functools.partial(
        pl.kernel,
        out_type=jax.ShapeDtypeStruct((NBLK, GB * D), jnp.float32),
        mesh=mesh,
        compiler_params=cp,
        scratch_types=[
            pltpu.VMEM((ROWS,), jnp.int32),
            pltpu.VMEM((ROWS,), jnp.int32),
            pltpu.VMEM((ROWS, D), jnp.float32),
            pltpu.VMEM((ROWS, D), jnp.float32),
            pltpu.VMEM((GB * D,), jnp.float32),
            pltpu.SemaphoreType.DMA,
        ],
    )
    def kern(table_hbm, ci_hbm, ni_hbm, out_hbm, ci_v, ni_v, cr_v, nr_v, ex_v,
             sem):
        wid = lax.axis_index("s") * 2 + lax.axis_index("c")

        @pl.loop(0, BLK_PER_W)
        def _(j):
            blk = wid * BLK_PER_W + j
            pltpu.sync_copy(ci_hbm.at[blk], ci_v)
            pltpu.sync_copy(ni_hbm.at[blk], ni_v)
            g1 = pltpu.async_copy(table_hbm.at[ci_v], cr_v, sem)
            g2 = pltpu.async_copy(table_hbm.at[ni_v], nr_v, sem)
            g1.wait()
            g2.wait()
            for bi in range(GB):
                cb0 = jnp.zeros((L,), jnp.float32)
                cb1 = jnp.zeros((L,), jnp.float32)
                for r in range(CTX):
                    row = bi * CTX + r
                    v0 = cr_v[row, pl.ds(0, L)]
                    v1 = cr_v[row, pl.ds(L, L)]
                    s = _row_scale(v0, v1)
                    cb0 = cb0 + v0 * s
                    cb1 = cb1 + v1 * s
                lane = lax.iota(jnp.int32, L)
                e_lo = jnp.zeros((L,), jnp.float32)
                e_hi = jnp.zeros((L,), jnp.float32)
                for r in range(PATH):
                    row = bi * PATH + r
                    v0 = nr_v[row, pl.ds(0, L)]
                    v1 = nr_v[row, pl.ds(L, L)]
                    s = _row_scale(v0, v1)
                    e = jnp.sum(v0 * cb0 + v1 * cb1) * s
                    if r < L:
                        e_lo = jnp.where(lane == r, e, e_lo)
                    else:
                        e_hi = jnp.where(lane == (r - L), e, e_hi)
                ex_v[pl.ds(bi * D, L)] = e_lo
                ex_v[pl.ds(bi * D + L, L)] = e_hi
            pltpu.sync_copy(ex_v, out_hbm.at[blk])

    return kern(table, ctx_idx, node_idx)


def _tc_loss(exc, turns, mask):
    def body(e_ref, t_ref, m_ref, o_ref):
        x = t_ref[...] * e_ref[:, :PATH]
        term = m_ref[...] / (1.0 + jnp.exp(-x))
        term = jnp.where(term == 0.0, 1.0, term)
        o_ref[0, 0] = -jnp.sum(jnp.log(term)) * (1.0 / B)

    return pl.pallas_call(
        body,
        out_shape=jax.ShapeDtypeStruct((1, 1), jnp.float32),
        out_specs=pl.BlockSpec(memory_space=pltpu.SMEM),
    )(exc, turns, mask)


def kernel(context, nodes, nodes_mask, turns_coeffs, table):
    ci = context.reshape(NBLK, ROWS)
    ni = nodes.reshape(NBLK, ROWS)
    exc = _sc_excitations(table, ci, ni)
    exc = exc.reshape(B, D)
    loss = _tc_loss(exc, turns_coeffs, nodes_mask)
    return loss[0, 0]


# speedup vs baseline: 1.8010x; 1.2020x over previous
import dataclasses
import functools

import jax
import jax.numpy as jnp
from jax import lax
from jax.experimental import pallas as pl
from jax.experimental.pallas import tpu as pltpu
from jax.experimental.pallas import tpu_sc as plsc

B = 16384
CTX = 20
PATH = 20
D = 32
L = 16
NW = 32
GB = 4
NBLK = B // GB
BLK_PER_W = NBLK // NW
ROWS = GB * CTX


def _rsqrt_nr(n2):
    i = lax.bitcast_convert_type(n2, jnp.int32)
    i = jnp.int32(0x5F3759DF) - lax.shift_right_logical(i, 1)
    y = lax.bitcast_convert_type(i, jnp.float32)
    for _ in range(3):
        y = y * (1.5 - 0.5 * n2 * y * y)
    return y


def _row_scale(v0, v1):
    n2 = jnp.sum(v0 * v0 + v1 * v1)
    return jnp.where(n2 > 1.0, _rsqrt_nr(n2), 1.0)


def _sc_excitations(table, ctx_idx, node_idx):
    mesh = plsc.VectorSubcoreMesh(core_axis_name="c", subcore_axis_name="s")
    cp = pltpu.CompilerParams()
    fields = pltpu.CompilerParams.__dataclass_fields__
    if "needs_layout_passes" in fields:
        cp = dataclasses.replace(cp, needs_layout_passes=False)
    if "use_tc_tiling_on_sc" in fields:
        cp = dataclasses.replace(cp, use_tc_tiling_on_sc=False)

    @functools.partial(
        pl.kernel,
        out_type=jax.ShapeDtypeStruct((NBLK, GB * D), jnp.float32),
        mesh=mesh,
        compiler_params=cp,
        scratch_types=[
            pltpu.VMEM((BLK_PER_W, ROWS), jnp.int32),
            pltpu.VMEM((BLK_PER_W, ROWS), jnp.int32),
            pltpu.VMEM((2, ROWS, D), jnp.float32),
            pltpu.VMEM((2, ROWS, D), jnp.float32),
            pltpu.VMEM((2, GB * D), jnp.float32),
            pltpu.SemaphoreType.DMA,
            pltpu.SemaphoreType.DMA,
            pltpu.SemaphoreType.DMA,
            pltpu.SemaphoreType.DMA,
        ],
    )
    def kern(table_hbm, ci_hbm, ni_hbm, out_hbm, ci_all, ni_all, cr, nr, ex,
             gs0, gs1, os0, os1):
        wid = lax.axis_index("s") * 2 + lax.axis_index("c")
        base = wid * BLK_PER_W
        gsem = (gs0, gs1)
        osem = (os0, os1)

        pltpu.sync_copy(ci_hbm.at[pl.ds(base, BLK_PER_W)], ci_all)
        pltpu.sync_copy(ni_hbm.at[pl.ds(base, BLK_PER_W)], ni_all)

        def start_gathers(b, p):
            pltpu.async_copy(table_hbm.at[ci_all.at[b]], cr.at[p], gsem[p])
            pltpu.async_copy(table_hbm.at[ni_all.at[b]], nr.at[p], gsem[p])

        def wait_gathers(b, p):
            pltpu.make_async_copy(table_hbm.at[ci_all.at[b]], cr.at[p],
                                  gsem[p]).wait()
            pltpu.make_async_copy(table_hbm.at[ni_all.at[b]], nr.at[p],
                                  gsem[p]).wait()

        start_gathers(0, 0)
        start_gathers(1, 1)

        @pl.loop(0, BLK_PER_W, step=2)
        def _(j):
          for p in range(2):
            b = j + p
            wait_gathers(b, p)
            cr_v = cr.at[p]
            nr_v = nr.at[p]
            ex_v = ex.at[p]
            @pl.when(b >= 2)
            def _():
                pltpu.make_async_copy(ex.at[p], out_hbm.at[base + b],
                                      osem[p]).wait()
            for bi in range(GB):
                cb0 = jnp.zeros((L,), jnp.float32)
                cb1 = jnp.zeros((L,), jnp.float32)
                for r in range(CTX):
                    row = bi * CTX + r
                    v0 = cr_v[row, pl.ds(0, L)]
                    v1 = cr_v[row, pl.ds(L, L)]
                    s = _row_scale(v0, v1)
                    cb0 = cb0 + v0 * s
                    cb1 = cb1 + v1 * s
                lane = lax.iota(jnp.int32, L)
                e_lo = jnp.zeros((L,), jnp.float32)
                e_hi = jnp.zeros((L,), jnp.float32)
                for r in range(PATH):
                    row = bi * PATH + r
                    v0 = nr_v[row, pl.ds(0, L)]
                    v1 = nr_v[row, pl.ds(L, L)]
                    s = _row_scale(v0, v1)
                    e = jnp.sum(v0 * cb0 + v1 * cb1) * s
                    if r < L:
                        e_lo = jnp.where(lane == r, e, e_lo)
                    else:
                        e_hi = jnp.where(lane == (r - L), e, e_hi)
                ex_v[pl.ds(bi * D, L)] = e_lo
                ex_v[pl.ds(bi * D + L, L)] = e_hi
            pltpu.async_copy(ex_v, out_hbm.at[base + b], osem[p])

            @pl.when(b + 2 < BLK_PER_W)
            def _():
                start_gathers(b + 2, p)

        for p in range(2):
            pltpu.make_async_copy(ex.at[p], out_hbm.at[base], osem[p]).wait()

    return kern(table, ctx_idx, node_idx)


def _tc_loss(exc, turns, mask):
    def body(e_ref, t_ref, m_ref, o_ref):
        x = t_ref[...] * e_ref[:, :PATH]
        term = m_ref[...] / (1.0 + jnp.exp(-x))
        term = jnp.where(term == 0.0, 1.0, term)
        o_ref[0, 0] = -jnp.sum(jnp.log(term)) * (1.0 / B)

    return pl.pallas_call(
        body,
        out_shape=jax.ShapeDtypeStruct((1, 1), jnp.float32),
        out_specs=pl.BlockSpec(memory_space=pltpu.SMEM),
    )(exc, turns, mask)


def kernel(context, nodes, nodes_mask, turns_coeffs, table):
    ci = context.reshape(NBLK, ROWS)
    ni = nodes.reshape(NBLK, ROWS)
    exc = _sc_excitations(table, ci, ni)
    exc = exc.reshape(B, D)
    loss = _tc_loss(exc, turns_coeffs, nodes_mask)
    return loss[0, 0]
